# baseline (speedup 1.0000x reference)
import functools

import jax
import jax.numpy as jnp
from jax import lax
from jax.experimental import pallas as pl
from jax.experimental.pallas import tpu as pltpu
from jax.experimental.pallas import tpu_sc as plsc

N = 10000
E = 320000
D = 128
LH = 64
G = 64
OUT = 64

NC = 2
NS = 16
CH = 128
NCHE = 160
EPT = CH * NCHE
EPAD = EPT * NS
ACC_ROWS = 10112
DEG_ROWS = 10240
CPO = 624
PGCH = 20
PGW = PGCH * CH
SEG_ROWS = 320
SEG_CH = 80


def _wid():
    return lax.axis_index("s") * NC + lax.axis_index("c")


@functools.lru_cache(maxsize=None)
def _mesh():
    return plsc.VectorSubcoreMesh(
        core_axis_name="c", subcore_axis_name="s",
        num_cores=NC, num_subcores=NS)


@functools.lru_cache(maxsize=None)
def _deg_kernel_call():
    return pl.kernel(
        _deg_body,
        out_type=jax.ShapeDtypeStruct((NC * DEG_ROWS,), jnp.float32),
        mesh=_mesh(),
        scratch_types=[
            pltpu.VMEM((EPT // 2,), jnp.int32),
            pltpu.VMEM((CH,), jnp.float32),
            pltpu.VMEM((640,), jnp.float32),
            pltpu.VMEM_SHARED((DEG_ROWS,), jnp.float32),
        ],
    )


def _deg_body(dst_hbm, out_hbm, dst_v, ones_v, z_v, acc_sh):
    c = lax.axis_index("c")
    s = lax.axis_index("s")

    one = jnp.full((16,), 1.0, dtype=jnp.float32)
    zero = jnp.zeros((16,), dtype=jnp.float32)

    def fill(i, _):
        ones_v[pl.ds(i * 16, 16)] = one
        return 0
    lax.fori_loop(0, CH // 16, fill, 0)

    def zfill(i, _):
        z_v[pl.ds(i * 16, 16)] = zero
        return 0
    lax.fori_loop(0, 640 // 16, zfill, 0)
    pltpu.sync_copy(z_v, acc_sh.at[pl.ds(s * 640, 640)])

    pltpu.sync_copy(dst_hbm.at[s].at[pl.ds(c * (EPT // 2), EPT // 2)], dst_v)
    plsc.subcore_barrier()

    def body(j, _):
        pltpu.sync_copy(ones_v, acc_sh.at[dst_v.at[pl.ds(j * CH, CH)]], add=True)
        return 0
    lax.fori_loop(0, NCHE // 2, body, 0)

    plsc.subcore_barrier()
    pltpu.sync_copy(acc_sh.at[pl.ds(s * 640, 640)],
                    out_hbm.at[pl.ds(c * DEG_ROWS + s * 640, 640)])


@functools.lru_cache(maxsize=None)
def _agg_kernel_call():
    return pl.kernel(
        _agg_body,
        out_type=jax.ShapeDtypeStruct((NC, N, LH), jnp.float32),
        mesh=_mesh(),
        scratch_types=[
            pltpu.VMEM((2 * PGW,), jnp.int32),
            pltpu.VMEM((2 * PGW,), jnp.int32),
            [pltpu.VMEM((CH, LH), jnp.float32) for _ in range(4)],
            [pltpu.SemaphoreType.DMA for _ in range(4)],
            [pltpu.SemaphoreType.DMA for _ in range(4)],
            [pltpu.SemaphoreType.DMA for _ in range(2)],
            pltpu.SemaphoreType.DMA,
            pltpu.VMEM_SHARED((ACC_ROWS, LH), jnp.float32),
            pltpu.VMEM_SHARED((N, LH), jnp.float32),
        ],
        compiler_params=pltpu.CompilerParams(use_tc_tiling_on_sc=False),
    )


def _agg_body(hs_hbm, src_hbm, dst_hbm, zero_hbm, out_hbm,
              src_v, dst_v, rows, gsems, ssems, pgsems, stgsem, acc_sh, hs_sp):
    c = lax.axis_index("c")
    s = lax.axis_index("s")
    hs_half = hs_hbm.at[c]
    idx_hbm = (src_hbm.at[s], dst_hbm.at[s])
    NB = 4
    LA = 2

    def _page_copies(p, slot):
        off = pl.ds(pl.multiple_of(p * PGW, PGW), PGW)
        dst = pl.ds(slot * PGW, PGW)
        return ((idx_hbm[0].at[off], src_v.at[dst], pgsems[slot]),
                (idx_hbm[1].at[off], dst_v.at[dst], pgsems[slot]))

    def load_page(p, slot):
        for args in _page_copies(p, slot):
            pltpu.async_copy(*args)

    def wait_page(p, slot):
        for args in _page_copies(p, slot):
            pltpu.make_async_copy(*args).wait()

    load_page(0, 0)
    load_page(1, 1)

    stg = pl.ds(s * CPO, CPO)
    pltpu.async_copy(hs_half.at[stg], hs_sp.at[stg], stgsem)

    pltpu.sync_copy(zero_hbm, acc_sh.at[pl.ds(s * (ACC_ROWS // NS), ACC_ROWS // NS)])

    @pl.when(s == NS - 1)
    def _stg_tail():
        tail = pl.ds(NS * CPO, N - NS * CPO)
        pltpu.sync_copy(hs_half.at[tail], hs_sp.at[tail])

    pltpu.make_async_copy(hs_half.at[stg], hs_sp.at[stg], stgsem).wait()
    wait_page(0, 0)
    plsc.subcore_barrier()

    def _sl(idx_v, j):
        return idx_v.at[pl.ds(pl.multiple_of((j % (2 * PGCH)) * CH, CH), CH)]

    def _gsrc(b):
        return hs_sp if b % 2 == 0 else hs_half

    def gather(j, b):
        pltpu.async_copy(_gsrc(b).at[_sl(src_v, j)], rows[b], gsems[b])

    def wait_gather(j, b):
        pltpu.make_async_copy(_gsrc(b).at[_sl(src_v, j)], rows[b], gsems[b]).wait()

    def scatter(j, b):
        pltpu.async_copy(rows[b], acc_sh.at[_sl(dst_v, j)], ssems[b], add=True)

    def wait_scatter(j, b):
        pltpu.make_async_copy(rows[b], acc_sh.at[_sl(dst_v, j)], ssems[b]).wait()

    for j in range(LA):
        gather(j, j)
    for j in range(NB):
        if j >= 2:
            wait_scatter(j - 2, j - 2)
        gather(j + LA, (j + LA) % NB)
        wait_gather(j, j)
        scatter(j, j)

    def body(jj, _):
        j0 = jj * NB
        for b in range(NB):
            j = j0 + b
            if b == 1:
                @pl.when(jnp.logical_and(jj % 10 == 4, jj <= 34))
                def _pw1():
                    wait_page((jj + 1) // 5, 1)

                @pl.when(jnp.logical_and(jj % 10 == 9, jj <= 29))
                def _pw0():
                    wait_page((jj + 1) // 5, 0)
            @pl.when(j + LA <= NCHE - 1)
            def _prefetch():
                wait_scatter(j - 2, (b + LA) % NB)
                gather(j + LA, (b + LA) % NB)
            if b == 2:
                @pl.when(jnp.logical_and(jj % 10 == 5, jj <= 25))
                def _pl0():
                    load_page(jj // 5 + 1, 0)

                @pl.when(jnp.logical_and(jj % 10 == 0, jj <= 30))
                def _pl1():
                    load_page(jj // 5 + 1, 1)
            wait_gather(j, b)
            scatter(j, b)
        return 0
    lax.fori_loop(1, NCHE // NB, body, 0)

    for j in range(NCHE - NB, NCHE):
        wait_scatter(j, j % NB)

    plsc.subcore_barrier()
    pltpu.sync_copy(acc_sh.at[pl.ds(s * CPO, CPO)],
                    out_hbm.at[c].at[pl.ds(s * CPO, CPO)])

    @pl.when(s == NS - 1)
    def _tail():
        pltpu.sync_copy(acc_sh.at[pl.ds(NS * CPO, N - NS * CPO)],
                        out_hbm.at[c].at[pl.ds(NS * CPO, N - NS * CPO)])


@functools.lru_cache(maxsize=None)
def _seg_kernel_call():
    return pl.kernel(
        _seg_body,
        out_type=jax.ShapeDtypeStruct((NC * NS * G * D,), jnp.float32),
        mesh=_mesh(),
        scratch_types=[
            pltpu.VMEM((SEG_CH * D,), jnp.float32),
            pltpu.VMEM((SEG_CH + 16,), jnp.int32),
            pltpu.VMEM((G * D,), jnp.float32),
        ],
    )


def _seg_body(jk_hbm, batch_hbm, out_hbm, jkc_v, batch_v, tab_v):
    wid = _wid()
    base = wid * SEG_ROWS
    minf = jnp.full((16,), -jnp.inf, dtype=jnp.float32)

    def ifill(i, _):
        tab_v[pl.ds(i * 16, 16)] = minf
        return 0
    lax.fori_loop(0, G * D // 16, ifill, 0)

    nrows = jnp.minimum(SEG_ROWS, N - base)
    nch = (nrows + SEG_CH - 1) // SEG_CH

    def chunk(j, carry):
        cb = base + j * SEG_CH
        pltpu.sync_copy(batch_hbm.at[pl.ds(cb, SEG_CH)], batch_v.at[pl.ds(0, SEG_CH)])
        pltpu.sync_copy(jk_hbm.at[pl.ds(cb * D, SEG_CH * D)], jkc_v)

        def row(r, carry):
            cur_g = carry[0]
            accs = carry[1:]
            g = batch_v[pl.ds(r, 16)][0]
            vals = tuple(jkc_v[pl.ds(r * D + k * 16, 16)] for k in range(8))
            is_new = g != cur_g

            @pl.when(jnp.logical_and(is_new, cur_g >= 0))
            def _flush():
                for k in range(8):
                    tab_v[pl.ds(cur_g * D + k * 16, 16)] = accs[k]

            new_accs = tuple(
                jnp.where(is_new, vals[k], jnp.maximum(accs[k], vals[k]))
                for k in range(8))
            return (g,) + new_accs

        return lax.fori_loop(0, SEG_CH, row, carry)

    init = (jnp.int32(-1),) + tuple(
        jnp.zeros((16,), dtype=jnp.float32) for _ in range(8))
    carry = lax.fori_loop(0, nch, chunk, init)

    cur_g = carry[0]
    accs = carry[1:]

    @pl.when(cur_g >= 0)
    def _final_flush():
        for k in range(8):
            tab_v[pl.ds(cur_g * D + k * 16, 16)] = accs[k]

    pltpu.sync_copy(tab_v, out_hbm.at[pl.ds(wid * G * D, G * D)])


_BLK = 1000
_GRID = N // _BLK


def _row_spec(shape):
    return pl.BlockSpec((_BLK,) + shape[1:], lambda i: (i,) + (0,) * (len(shape) - 1))


def _full_spec(shape):
    return pl.BlockSpec(shape, lambda i: (0,) * len(shape))


def _half_spec():
    return pl.BlockSpec((NC, _BLK, LH), lambda i: (0, i, 0))


def _cat(h_ref):
    return jnp.concatenate([h_ref[0], h_ref[1]], axis=-1)


def _mm1_body(x_ref, w_ref, dis_ref, hs_ref):
    hs = jnp.dot(x_ref[...], w_ref[...],
                 preferred_element_type=jnp.float32) * dis_ref[...]
    hs_ref[0] = hs[:, :LH]
    hs_ref[1] = hs[:, LH:]


def _mm1(x, W, dis_b):
    return pl.pallas_call(
        _mm1_body,
        grid=(_GRID,),
        in_specs=[_row_spec((N, D)), _full_spec((D, D)), _row_spec((N, D))],
        out_specs=_half_spec(),
        out_shape=jax.ShapeDtypeStruct((NC, N, LH), jnp.float32),
    )(x, W, dis_b)


def _mm2_body(agg_ref, hs_ref, dis_ref, b_ref, w_ref, y_ref, hsn_ref):
    comb = (_cat(agg_ref) + _cat(hs_ref)) * dis_ref[...] + b_ref[...]
    y = jnp.maximum(comb, 0.0)
    y_ref[...] = y
    hsn = jnp.dot(y, w_ref[...],
                  preferred_element_type=jnp.float32) * dis_ref[...]
    hsn_ref[0] = hsn[:, :LH]
    hsn_ref[1] = hsn[:, LH:]


def _mm2(aggp, hs, dis_b, b, Wn):
    return pl.pallas_call(
        _mm2_body,
        grid=(_GRID,),
        in_specs=[_half_spec(), _half_spec(), _row_spec((N, D)),
                  _full_spec((1, D)), _full_spec((D, D))],
        out_specs=[_row_spec((N, D)), _half_spec()],
        out_shape=[jax.ShapeDtypeStruct((N, D), jnp.float32),
                   jax.ShapeDtypeStruct((NC, N, LH), jnp.float32)],
    )(aggp, hs, dis_b, b, Wn)


def _jk_body(agg_ref, hs_ref, dis_ref, b_ref, y1_ref, y2_ref, jk_ref):
    comb = (_cat(agg_ref) + _cat(hs_ref)) * dis_ref[...] + b_ref[...]
    y3 = jnp.maximum(comb, 0.0)
    jk_ref[...] = jnp.maximum(jnp.maximum(y1_ref[...], y2_ref[...]), y3)


def _jk(aggp, hs, dis_b, b, y1, y2):
    return pl.pallas_call(
        _jk_body,
        grid=(_GRID,),
        in_specs=[_half_spec(), _half_spec(), _row_spec((N, D)),
                  _full_spec((1, D)), _row_spec((N, D)), _row_spec((N, D))],
        out_specs=_row_spec((N, D)),
        out_shape=jax.ShapeDtypeStruct((N, D), jnp.float32),
    )(aggp, hs, dis_b, b, y1, y2)


def _final_body(segp_ref, lw_ref, lb_ref, ow_ref, ob_ref, out_ref):
    g = jnp.max(segp_ref[...], axis=0)
    a = jnp.maximum(jnp.dot(g, lw_ref[...],
                            preferred_element_type=jnp.float32) + lb_ref[...], 0.0)
    out_ref[...] = jnp.dot(a, ow_ref[...],
                           preferred_element_type=jnp.float32) + ob_ref[...]


def _final(segp, lin_W, lin_b, out_W, out_b):
    return pl.pallas_call(
        _final_body,
        out_shape=jax.ShapeDtypeStruct((G, OUT), jnp.float32),
    )(segp, lin_W, lin_b.reshape(1, D), out_W, out_b.reshape(1, OUT))


def kernel(x, edge_index, batch, pos, W0, b0, W1, b1, W2, b2,
           lin_W, lin_b, out_W, out_b):
    del pos
    src = edge_index[0]
    dst = edge_index[1]
    npad = EPAD - E
    pad8 = (jnp.arange(npad, dtype=jnp.int32) % 8)
    src_p = jnp.concatenate([src, pad8]).reshape(NS, EPT)
    dst_p = jnp.concatenate([dst, N + pad8]).reshape(NS, EPT)

    degp = _deg_kernel_call()(dst_p).reshape(NC, DEG_ROWS)
    deg = degp[0, :N] + degp[1, :N] + 1.0
    dis_b = jnp.broadcast_to(lax.rsqrt(deg)[:, None], (N, D))

    agg = _agg_kernel_call()
    zero_page = jnp.zeros((ACC_ROWS // NS, LH), jnp.float32)
    hs1 = _mm1(x, W0, dis_b)
    a1 = agg(hs1, src_p, dst_p, zero_page)
    y1, hs2 = _mm2(a1, hs1, dis_b, b0.reshape(1, D), W1)
    a2 = agg(hs2, src_p, dst_p, zero_page)
    y2, hs3 = _mm2(a2, hs2, dis_b, b1.reshape(1, D), W2)
    a3 = agg(hs3, src_p, dst_p, zero_page)
    jk = _jk(a3, hs3, dis_b, b2.reshape(1, D), y1, y2)

    segp = _seg_kernel_call()(jk.reshape(N * D), batch).reshape(NC * NS, G, D)
    return _final(segp, lin_W, lin_b, out_W, out_b)


# speedup vs baseline: 1.1902x; 1.1902x over previous
import functools

import jax
import jax.numpy as jnp
from jax import lax
from jax.experimental import pallas as pl
from jax.experimental.pallas import tpu as pltpu
from jax.experimental.pallas import tpu_sc as plsc

N = 10000
E = 320000
D = 128
LH = 64
G = 64
OUT = 64

NC = 2
NS = 16
CH = 128
NCHE = 160
EPT = CH * NCHE
EPAD = EPT * NS
ACC_ROWS = 10112
DEG_ROWS = 10240
CPO = 624
PGCH = 20
PGW = PGCH * CH
SEG_ROWS = 320
SEG_CH = 80


def _wid():
    return lax.axis_index("s") * NC + lax.axis_index("c")


@functools.lru_cache(maxsize=None)
def _mesh():
    return plsc.VectorSubcoreMesh(
        core_axis_name="c", subcore_axis_name="s",
        num_cores=NC, num_subcores=NS)


@functools.lru_cache(maxsize=None)
def _deg_kernel_call():
    return pl.kernel(
        _deg_body,
        out_type=jax.ShapeDtypeStruct((NC * DEG_ROWS,), jnp.float32),
        mesh=_mesh(),
        scratch_types=[
            pltpu.VMEM((EPT // 2,), jnp.int32),
            pltpu.VMEM((CH,), jnp.float32),
            pltpu.VMEM((640,), jnp.float32),
            pltpu.VMEM_SHARED((DEG_ROWS,), jnp.float32),
        ],
    )


def _deg_body(dst_hbm, out_hbm, dst_v, ones_v, z_v, acc_sh):
    c = lax.axis_index("c")
    s = lax.axis_index("s")

    one = jnp.full((16,), 1.0, dtype=jnp.float32)
    zero = jnp.zeros((16,), dtype=jnp.float32)

    def fill(i, _):
        ones_v[pl.ds(i * 16, 16)] = one
        return 0
    lax.fori_loop(0, CH // 16, fill, 0)

    def zfill(i, _):
        z_v[pl.ds(i * 16, 16)] = zero
        return 0
    lax.fori_loop(0, 640 // 16, zfill, 0)
    pltpu.sync_copy(z_v, acc_sh.at[pl.ds(s * 640, 640)])

    pltpu.sync_copy(dst_hbm.at[s].at[pl.ds(c * (EPT // 2), EPT // 2)], dst_v)
    plsc.subcore_barrier()

    def body(j, _):
        pltpu.sync_copy(ones_v, acc_sh.at[dst_v.at[pl.ds(j * CH, CH)]], add=True)
        return 0
    lax.fori_loop(0, NCHE // 2, body, 0)

    plsc.subcore_barrier()
    pltpu.sync_copy(acc_sh.at[pl.ds(s * 640, 640)],
                    out_hbm.at[pl.ds(c * DEG_ROWS + s * 640, 640)])


@functools.lru_cache(maxsize=None)
def _agg_kernel_call():
    return pl.kernel(
        _agg_body,
        out_type=jax.ShapeDtypeStruct((NC, N, LH), jnp.float32),
        mesh=_mesh(),
        scratch_types=[
            pltpu.VMEM((2 * PGW,), jnp.int32),
            pltpu.VMEM((2 * PGW,), jnp.int32),
            [pltpu.VMEM((CH, LH), jnp.float32) for _ in range(4)],
            [pltpu.SemaphoreType.DMA for _ in range(4)],
            [pltpu.SemaphoreType.DMA for _ in range(4)],
            [pltpu.SemaphoreType.DMA for _ in range(2)],
            pltpu.SemaphoreType.DMA,
            pltpu.VMEM_SHARED((ACC_ROWS, LH), jnp.float32),
            pltpu.VMEM_SHARED((N, LH), jnp.float32),
        ],
        compiler_params=pltpu.CompilerParams(use_tc_tiling_on_sc=False),
    )


def _agg_body(hs_hbm, src_hbm, dst_hbm, zero_hbm, out_hbm,
              src_v, dst_v, rows, gsems, ssems, pgsems, stgsem, acc_sh, hs_sp):
    c = lax.axis_index("c")
    s = lax.axis_index("s")
    hs_half = hs_hbm.at[c]
    idx_hbm = (src_hbm.at[s], dst_hbm.at[s])
    NB = 4
    LA = 2

    def _page_copies(p, slot):
        off = pl.ds(pl.multiple_of(p * PGW, PGW), PGW)
        dst = pl.ds(slot * PGW, PGW)
        return ((idx_hbm[0].at[off], src_v.at[dst], pgsems[slot]),
                (idx_hbm[1].at[off], dst_v.at[dst], pgsems[slot]))

    def load_page(p, slot):
        for args in _page_copies(p, slot):
            pltpu.async_copy(*args)

    def wait_page(p, slot):
        for args in _page_copies(p, slot):
            pltpu.make_async_copy(*args).wait()

    load_page(0, 0)
    load_page(1, 1)

    stg = pl.ds(s * CPO, CPO)
    pltpu.async_copy(hs_half.at[stg], hs_sp.at[stg], stgsem)

    pltpu.sync_copy(zero_hbm, acc_sh.at[pl.ds(s * (ACC_ROWS // NS), ACC_ROWS // NS)])

    @pl.when(s == NS - 1)
    def _stg_tail():
        tail = pl.ds(NS * CPO, N - NS * CPO)
        pltpu.sync_copy(hs_half.at[tail], hs_sp.at[tail])

    pltpu.make_async_copy(hs_half.at[stg], hs_sp.at[stg], stgsem).wait()
    wait_page(0, 0)
    plsc.subcore_barrier()

    def _sl(idx_v, j):
        return idx_v.at[pl.ds(pl.multiple_of((j % (2 * PGCH)) * CH, CH), CH)]

    def gather(j, b):
        pltpu.async_copy(hs_sp.at[_sl(src_v, j)], rows[b], gsems[b])

    def wait_gather(j, b):
        pltpu.make_async_copy(hs_sp.at[_sl(src_v, j)], rows[b], gsems[b]).wait()

    def scatter(j, b):
        pltpu.async_copy(rows[b], acc_sh.at[_sl(dst_v, j)], ssems[b], add=True)

    def wait_scatter(j, b):
        pltpu.make_async_copy(rows[b], acc_sh.at[_sl(dst_v, j)], ssems[b]).wait()

    for j in range(LA):
        gather(j, j)
    for j in range(NB):
        if j >= 2:
            wait_scatter(j - 2, j - 2)
        gather(j + LA, (j + LA) % NB)
        wait_gather(j, j)
        scatter(j, j)

    def body(jj, _):
        j0 = jj * NB
        for b in range(NB):
            j = j0 + b
            if b == 1:
                @pl.when(jnp.logical_and(jj % 10 == 4, jj <= 34))
                def _pw1():
                    wait_page((jj + 1) // 5, 1)

                @pl.when(jnp.logical_and(jj % 10 == 9, jj <= 29))
                def _pw0():
                    wait_page((jj + 1) // 5, 0)
            @pl.when(j + LA <= NCHE - 1)
            def _prefetch():
                wait_scatter(j - 2, (b + LA) % NB)
                gather(j + LA, (b + LA) % NB)
            if b == 2:
                @pl.when(jnp.logical_and(jj % 10 == 5, jj <= 25))
                def _pl0():
                    load_page(jj // 5 + 1, 0)

                @pl.when(jnp.logical_and(jj % 10 == 0, jj <= 30))
                def _pl1():
                    load_page(jj // 5 + 1, 1)
            wait_gather(j, b)
            scatter(j, b)
        return 0
    lax.fori_loop(1, NCHE // NB, body, 0)

    for j in range(NCHE - NB, NCHE):
        wait_scatter(j, j % NB)

    plsc.subcore_barrier()
    pltpu.sync_copy(acc_sh.at[pl.ds(s * CPO, CPO)],
                    out_hbm.at[c].at[pl.ds(s * CPO, CPO)])

    @pl.when(s == NS - 1)
    def _tail():
        pltpu.sync_copy(acc_sh.at[pl.ds(NS * CPO, N - NS * CPO)],
                        out_hbm.at[c].at[pl.ds(NS * CPO, N - NS * CPO)])


@functools.lru_cache(maxsize=None)
def _seg_kernel_call():
    return pl.kernel(
        _seg_body,
        out_type=jax.ShapeDtypeStruct((NC * NS * G * D,), jnp.float32),
        mesh=_mesh(),
        scratch_types=[
            pltpu.VMEM((SEG_CH * D,), jnp.float32),
            pltpu.VMEM((SEG_CH + 16,), jnp.int32),
            pltpu.VMEM((G * D,), jnp.float32),
        ],
    )


def _seg_body(jk_hbm, batch_hbm, out_hbm, jkc_v, batch_v, tab_v):
    wid = _wid()
    base = wid * SEG_ROWS
    minf = jnp.full((16,), -jnp.inf, dtype=jnp.float32)

    def ifill(i, _):
        tab_v[pl.ds(i * 16, 16)] = minf
        return 0
    lax.fori_loop(0, G * D // 16, ifill, 0)

    nrows = jnp.minimum(SEG_ROWS, N - base)
    nch = (nrows + SEG_CH - 1) // SEG_CH

    def chunk(j, carry):
        cb = base + j * SEG_CH
        pltpu.sync_copy(batch_hbm.at[pl.ds(cb, SEG_CH)], batch_v.at[pl.ds(0, SEG_CH)])
        pltpu.sync_copy(jk_hbm.at[pl.ds(cb * D, SEG_CH * D)], jkc_v)

        def row(r, carry):
            cur_g = carry[0]
            accs = carry[1:]
            g = batch_v[pl.ds(r, 16)][0]
            vals = tuple(jkc_v[pl.ds(r * D + k * 16, 16)] for k in range(8))
            is_new = g != cur_g

            @pl.when(jnp.logical_and(is_new, cur_g >= 0))
            def _flush():
                for k in range(8):
                    tab_v[pl.ds(cur_g * D + k * 16, 16)] = accs[k]

            new_accs = tuple(
                jnp.where(is_new, vals[k], jnp.maximum(accs[k], vals[k]))
                for k in range(8))
            return (g,) + new_accs

        return lax.fori_loop(0, SEG_CH, row, carry)

    init = (jnp.int32(-1),) + tuple(
        jnp.zeros((16,), dtype=jnp.float32) for _ in range(8))
    carry = lax.fori_loop(0, nch, chunk, init)

    cur_g = carry[0]
    accs = carry[1:]

    @pl.when(cur_g >= 0)
    def _final_flush():
        for k in range(8):
            tab_v[pl.ds(cur_g * D + k * 16, 16)] = accs[k]

    pltpu.sync_copy(tab_v, out_hbm.at[pl.ds(wid * G * D, G * D)])


_BLK = 1000
_GRID = N // _BLK


def _row_spec(shape):
    return pl.BlockSpec((_BLK,) + shape[1:], lambda i: (i,) + (0,) * (len(shape) - 1))


def _full_spec(shape):
    return pl.BlockSpec(shape, lambda i: (0,) * len(shape))


def _half_spec():
    return pl.BlockSpec((NC, _BLK, LH), lambda i: (0, i, 0))


def _cat(h_ref):
    return jnp.concatenate([h_ref[0], h_ref[1]], axis=-1)


def _mm1_body(x_ref, w_ref, dis_ref, hs_ref):
    hs = jnp.dot(x_ref[...], w_ref[...],
                 preferred_element_type=jnp.float32) * dis_ref[...]
    hs_ref[0] = hs[:, :LH]
    hs_ref[1] = hs[:, LH:]


def _mm1(x, W, dis_b):
    return pl.pallas_call(
        _mm1_body,
        grid=(_GRID,),
        in_specs=[_row_spec((N, D)), _full_spec((D, D)), _row_spec((N, D))],
        out_specs=_half_spec(),
        out_shape=jax.ShapeDtypeStruct((NC, N, LH), jnp.float32),
    )(x, W, dis_b)


def _mm2_body(agg_ref, hs_ref, dis_ref, b_ref, w_ref, y_ref, hsn_ref):
    comb = (_cat(agg_ref) + _cat(hs_ref)) * dis_ref[...] + b_ref[...]
    y = jnp.maximum(comb, 0.0)
    y_ref[...] = y
    hsn = jnp.dot(y, w_ref[...],
                  preferred_element_type=jnp.float32) * dis_ref[...]
    hsn_ref[0] = hsn[:, :LH]
    hsn_ref[1] = hsn[:, LH:]


def _mm2(aggp, hs, dis_b, b, Wn):
    return pl.pallas_call(
        _mm2_body,
        grid=(_GRID,),
        in_specs=[_half_spec(), _half_spec(), _row_spec((N, D)),
                  _full_spec((1, D)), _full_spec((D, D))],
        out_specs=[_row_spec((N, D)), _half_spec()],
        out_shape=[jax.ShapeDtypeStruct((N, D), jnp.float32),
                   jax.ShapeDtypeStruct((NC, N, LH), jnp.float32)],
    )(aggp, hs, dis_b, b, Wn)


def _jk_body(agg_ref, hs_ref, dis_ref, b_ref, y1_ref, y2_ref, jk_ref):
    comb = (_cat(agg_ref) + _cat(hs_ref)) * dis_ref[...] + b_ref[...]
    y3 = jnp.maximum(comb, 0.0)
    jk_ref[...] = jnp.maximum(jnp.maximum(y1_ref[...], y2_ref[...]), y3)


def _jk(aggp, hs, dis_b, b, y1, y2):
    return pl.pallas_call(
        _jk_body,
        grid=(_GRID,),
        in_specs=[_half_spec(), _half_spec(), _row_spec((N, D)),
                  _full_spec((1, D)), _row_spec((N, D)), _row_spec((N, D))],
        out_specs=_row_spec((N, D)),
        out_shape=jax.ShapeDtypeStruct((N, D), jnp.float32),
    )(aggp, hs, dis_b, b, y1, y2)


def _final_body(segp_ref, lw_ref, lb_ref, ow_ref, ob_ref, out_ref):
    g = jnp.max(segp_ref[...], axis=0)
    a = jnp.maximum(jnp.dot(g, lw_ref[...],
                            preferred_element_type=jnp.float32) + lb_ref[...], 0.0)
    out_ref[...] = jnp.dot(a, ow_ref[...],
                           preferred_element_type=jnp.float32) + ob_ref[...]


def _final(segp, lin_W, lin_b, out_W, out_b):
    return pl.pallas_call(
        _final_body,
        out_shape=jax.ShapeDtypeStruct((G, OUT), jnp.float32),
    )(segp, lin_W, lin_b.reshape(1, D), out_W, out_b.reshape(1, OUT))


def kernel(x, edge_index, batch, pos, W0, b0, W1, b1, W2, b2,
           lin_W, lin_b, out_W, out_b):
    del pos
    src = edge_index[0]
    dst = edge_index[1]
    npad = EPAD - E
    pad8 = (jnp.arange(npad, dtype=jnp.int32) % 8)
    src_p = jnp.concatenate([src, pad8]).reshape(NS, EPT)
    dst_p = jnp.concatenate([dst, N + pad8]).reshape(NS, EPT)

    degp = _deg_kernel_call()(dst_p).reshape(NC, DEG_ROWS)
    deg = degp[0, :N] + degp[1, :N] + 1.0
    dis_b = jnp.broadcast_to(lax.rsqrt(deg)[:, None], (N, D))

    agg = _agg_kernel_call()
    zero_page = jnp.zeros((ACC_ROWS // NS, LH), jnp.float32)
    hs1 = _mm1(x, W0, dis_b)
    a1 = agg(hs1, src_p, dst_p, zero_page)
    y1, hs2 = _mm2(a1, hs1, dis_b, b0.reshape(1, D), W1)
    a2 = agg(hs2, src_p, dst_p, zero_page)
    y2, hs3 = _mm2(a2, hs2, dis_b, b1.reshape(1, D), W2)
    a3 = agg(hs3, src_p, dst_p, zero_page)
    jk = _jk(a3, hs3, dis_b, b2.reshape(1, D), y1, y2)

    segp = _seg_kernel_call()(jk.reshape(N * D), batch).reshape(NC * NS, G, D)
    return _final(segp, lin_W, lin_b, out_W, out_b)
